# baseline (speedup 1.0000x reference)
import functools

import jax
import jax.numpy as jnp
from jax import lax
from jax.experimental import pallas as pl
from jax.experimental.pallas import tpu as pltpu
from jax.experimental.pallas import tpu_sc as plsc

T = 512
ROW = 3 * 224 * 224
NKEY = 171
NW = 32
PER_W = -(-NKEY // NW)


def kernel(video):
    v2 = video.reshape(T, ROW)
    mesh = plsc.VectorSubcoreMesh(core_axis_name="c", subcore_axis_name="s")

    @functools.partial(
        pl.kernel,
        mesh=mesh,
        out_type=jax.ShapeDtypeStruct((NKEY, ROW), jnp.float32),
    )
    def k(v_hbm, o_hbm):
        wid = lax.axis_index("s") * 2 + lax.axis_index("c")
        for j in range(PER_W):
            f = j * NW + wid

            @pl.when(f < NKEY)
            def _():
                src = jnp.maximum(3 * f - 1, 0)
                pltpu.sync_copy(v_hbm.at[src], o_hbm.at[f])

    out = k(v2)
    return out.reshape(NKEY, 3, 224, 224)
